# baseline (speedup 1.0000x reference)
import numpy as np
import jax
import jax.numpy as jnp
from jax.experimental import pallas as pl
from jax.experimental.pallas import tpu as pltpu

_B, _V = 16384, 130
_W = _V * 3
_F = _V - 2
_COUNT = 3 * _F * _B
_LANES = 512
_NFLAT = _B * _W
_ROWS = _NFLAT // _LANES
_BR = 1560
_GRID = _ROWS // _BR


def _make_masks():
    j = np.arange(_BR * _LANES, dtype=np.int64) % _W
    tri = j % 3 == 0
    wn = np.where(tri & (j <= 384), np.where((j == 0) | (j == 384), 1.0, 2.0), 0.0)
    ws = np.where(tri & (j <= 381), 1.0, 0.0)
    scale = 1.0 / _COUNT
    return (
        (wn * scale).reshape(_BR, _LANES).astype(np.float32),
        (ws * scale).reshape(_BR, _LANES).astype(np.float32),
    )


_WN, _WS = _make_masks()


def _body(xo_ref, xg_ref, wn_ref, ws_ref, o_ref):
    pid = pl.program_id(0)

    @pl.when(pid == 0)
    def _():
        o_ref[0, 0] = 0.0

    lane = jax.lax.broadcasted_iota(jnp.int32, (_BR, _LANES), 1)

    def flatroll(v, k):
        a = jnp.roll(v, -k, axis=1)
        b = jnp.roll(a, -1, axis=0)
        return jnp.where(lane < _LANES - k, a, b)

    def edge_dists(x):
        e = flatroll(x, 3) - x
        e2 = e * e
        n2 = e2 + flatroll(e2, 1) + flatroll(e2, 2)
        f = flatroll(x, 6) - x
        f2 = f * f
        s2 = f2 + flatroll(f2, 1) + flatroll(f2, 2)
        return jnp.sqrt(n2), jnp.sqrt(s2)

    no, so = edge_dists(xo_ref[...])
    ng, sg = edge_dists(xg_ref[...])
    tot = jnp.abs(no - ng) * wn_ref[...] + jnp.abs(so - sg) * ws_ref[...]
    o_ref[0, 0] += jnp.sum(tot)


@jax.jit
def kernel(coord_out, coord_gt):
    xo = coord_out.reshape(_ROWS, _LANES)
    xg = coord_gt.reshape(_ROWS, _LANES)
    acc = pl.pallas_call(
        _body,
        grid=(_GRID,),
        in_specs=[
            pl.BlockSpec((_BR, _LANES), lambda i: (i, 0)),
            pl.BlockSpec((_BR, _LANES), lambda i: (i, 0)),
            pl.BlockSpec((_BR, _LANES), lambda i: (0, 0)),
            pl.BlockSpec((_BR, _LANES), lambda i: (0, 0)),
        ],
        out_specs=pl.BlockSpec(memory_space=pltpu.SMEM),
        out_shape=jax.ShapeDtypeStruct((1, 1), jnp.float32),
        compiler_params=pltpu.CompilerParams(
            dimension_semantics=("arbitrary",)),
    )(xo, xg, _WN, _WS)
    return acc[0, 0]


# speedup vs baseline: 254.3367x; 254.3367x over previous
import jax
import jax.numpy as jnp
from jax.experimental import pallas as pl
from jax.experimental.pallas import tpu as pltpu

_B, _V = 16384, 130
_F = _V - 2
_COUNT = 3 * _F * _B
_BL = 2048
_GRID = _B // _BL


def _body(xo_ref, xg_ref, o_ref):
    pid = pl.program_id(0)

    @pl.when(pid == 0)
    def _():
        o_ref[0, 0] = 0.0

    def edge_dists(x):
        e = x[:, 1:, :] - x[:, :-1, :]
        e2 = e * e
        n2 = e2[0] + e2[1] + e2[2]
        f = x[:, 2:, :] - x[:, :-2, :]
        f2 = f * f
        s2 = f2[0] + f2[1] + f2[2]
        return jnp.sqrt(n2), jnp.sqrt(s2)

    no, so = edge_dists(xo_ref[...])
    ng, sg = edge_dists(xg_ref[...])
    dn = jnp.abs(no - ng)
    ds = jnp.abs(so - sg)
    part = (2.0 * jnp.sum(dn) - jnp.sum(dn[0]) - jnp.sum(dn[128])
            + jnp.sum(ds))
    o_ref[0, 0] += part * (1.0 / _COUNT)


@jax.jit
def kernel(coord_out, coord_gt):
    xo = coord_out.transpose(2, 1, 0)
    xg = coord_gt.transpose(2, 1, 0)
    acc = pl.pallas_call(
        _body,
        grid=(_GRID,),
        in_specs=[
            pl.BlockSpec((3, _V, _BL), lambda i: (0, 0, i)),
            pl.BlockSpec((3, _V, _BL), lambda i: (0, 0, i)),
        ],
        out_specs=pl.BlockSpec(memory_space=pltpu.SMEM),
        out_shape=jax.ShapeDtypeStruct((1, 1), jnp.float32),
        compiler_params=pltpu.CompilerParams(
            dimension_semantics=("arbitrary",)),
    )(xo, xg)
    return acc[0, 0]
